# baseline (speedup 1.0000x reference)
import jax
import jax.numpy as jnp
from jax import lax
from jax.experimental import pallas as pl
from jax.experimental.pallas import tpu as pltpu
from jax.experimental.pallas import tpu_sc as plsc

BATCH = 4096
SLOTS = 8
KD = 64
LANES = 16
NCHUNK = KD // LANES
NC = 2
NS = 16
NW = NC * NS
UPW = BATCH // NW
CH = 32
NCB = UPW // CH


def _mann_body(user_hbm, query_hbm, value_hbm, out_hbm,
               idx_v, rows_a, rows_b, q_v, out_v, sem_a, sem_b):
    wid = lax.axis_index("s") * NC + lax.axis_index("c")
    base = wid * UPW

    pltpu.sync_copy(user_hbm.at[pl.ds(base, UPW)], idx_v.at[pl.ds(0, UPW)])

    rows = [rows_a, rows_b]
    sems = [sem_a, sem_b]

    def fire(cb):
        sem = sems[cb % 2]
        dst = rows[cb % 2]
        copies = []
        for t in range(CH):
            uid = idx_v[pl.ds(cb * CH + t, LANES)][0]
            copies.append(pltpu.async_copy(value_hbm.at[uid], dst.at[t], sem))
        return copies

    lanes = lax.iota(jnp.int32, LANES)
    slot_masks = [lanes == s for s in range(SLOTS)]
    neg_fill = jnp.full((LANES,), -1e30, jnp.float32)

    def make_user_body(rows_v, cb):
        def user_body(t, carry):
            u = cb * CH + t
            q = [q_v[u, pl.ds(c * LANES, LANES)] for c in range(NCHUNK)]
            mk = [[rows_v[t, s, pl.ds(c * LANES, LANES)] for c in range(NCHUNK)]
                  for s in range(SLOTS)]

            sv = neg_fill
            for s in range(SLOTS):
                acc = mk[s][0] * q[0]
                for c in range(1, NCHUNK):
                    acc = acc + mk[s][c] * q[c]
                sv = jnp.where(slot_masks[s], jnp.sum(acc), sv)

            m = jnp.max(sv)
            e = jnp.exp(sv - m)
            denom = jnp.sum(e)

            eb = [jnp.sum(jnp.where(slot_masks[s], e, 0.0))
                  for s in range(SLOTS)]
            for c in range(NCHUNK):
                acc = eb[0] * mk[0][c]
                for s in range(1, SLOTS):
                    acc = acc + eb[s] * mk[s][c]
                out_v[u, pl.ds(c * LANES, LANES)] = acc / denom
            return carry
        return user_body

    pltpu.sync_copy(query_hbm.at[pl.ds(base, UPW)], q_v)

    inflight = fire(0)
    for cb in range(NCB):
        nxt = fire(cb + 1) if cb + 1 < NCB else []
        for cp in inflight:
            cp.wait()
        lax.fori_loop(0, CH, make_user_body(rows[cb % 2], cb), 0)
        inflight = nxt

    pltpu.sync_copy(out_v, out_hbm.at[pl.ds(base, UPW)])


def kernel(user, query, Value):
    mesh = plsc.VectorSubcoreMesh(core_axis_name="c", subcore_axis_name="s")
    run = pl.kernel(
        _mann_body,
        out_type=jax.ShapeDtypeStruct((BATCH, KD), jnp.float32),
        mesh=mesh,
        compiler_params=pltpu.CompilerParams(needs_layout_passes=False),
        scratch_types=[
            pltpu.VMEM((UPW + LANES,), jnp.int32),
            pltpu.VMEM((CH, SLOTS, KD), jnp.float32),
            pltpu.VMEM((CH, SLOTS, KD), jnp.float32),
            pltpu.VMEM((UPW, KD), jnp.float32),
            pltpu.VMEM((UPW, KD), jnp.float32),
            pltpu.SemaphoreType.DMA,
            pltpu.SemaphoreType.DMA,
        ],
    )
    return run(user.astype(jnp.int32), query, Value)


# speedup vs baseline: 1.4489x; 1.4489x over previous
import jax
import jax.numpy as jnp
from jax import lax
from jax.experimental import pallas as pl
from jax.experimental.pallas import tpu as pltpu
from jax.experimental.pallas import tpu_sc as plsc

BATCH = 4096
SLOTS = 8
KD = 64
RD = SLOTS * KD
LANES = 16
NCHUNK = KD // LANES
NC = 2
NS = 16
NW = NC * NS
UPW = BATCH // NW
CH = 32
NCB = UPW // CH


def _mann_body(user_hbm, query_hbm, value_hbm, out_hbm,
               idx_v, rows_a, rows_b, q_v, out_v, sem_a, sem_b):
    wid = lax.axis_index("s") * NC + lax.axis_index("c")
    base = wid * UPW

    pltpu.sync_copy(user_hbm.at[pl.ds(base, UPW)], idx_v)

    rows = [rows_a, rows_b]
    sems = [sem_a, sem_b]

    def fire(cb):
        return pltpu.async_copy(
            value_hbm.at[idx_v.at[pl.ds(cb * CH, CH)]],
            rows[cb % 2], sems[cb % 2])

    lanes = lax.iota(jnp.int32, LANES)
    slot_masks = [lanes == s for s in range(SLOTS)]
    neg_fill = jnp.full((LANES,), -1e30, jnp.float32)

    def make_user_body(rows_v, cb):
        def user_body(t):
            u = cb * CH + t
            q = [q_v[u, pl.ds(c * LANES, LANES)] for c in range(NCHUNK)]
            mk = [[rows_v[t, pl.ds(s * KD + c * LANES, LANES)]
                   for c in range(NCHUNK)] for s in range(SLOTS)]

            sv = neg_fill
            for s in range(SLOTS):
                acc = mk[s][0] * q[0]
                for c in range(1, NCHUNK):
                    acc = acc + mk[s][c] * q[c]
                sv = jnp.where(slot_masks[s], jnp.sum(acc), sv)

            m = jnp.max(sv)
            e = jnp.exp(sv - m)
            denom = jnp.sum(e)

            eb = [jnp.sum(jnp.where(slot_masks[s], e, 0.0))
                  for s in range(SLOTS)]
            for c in range(NCHUNK):
                acc = eb[0] * mk[0][c]
                for s in range(1, SLOTS):
                    acc = acc + eb[s] * mk[s][c]
                out_v[u, pl.ds(c * LANES, LANES)] = acc / denom
        return user_body

    pltpu.sync_copy(query_hbm.at[pl.ds(base, UPW)], q_v)

    inflight = fire(0)
    for cb in range(NCB):
        nxt = fire(cb + 1) if cb + 1 < NCB else None
        inflight.wait()
        plsc.parallel_loop(0, CH, unroll=2)(make_user_body(rows[cb % 2], cb))
        inflight = nxt

    pltpu.sync_copy(out_v, out_hbm.at[pl.ds(base, UPW)])


def kernel(user, query, Value):
    mesh = plsc.VectorSubcoreMesh(core_axis_name="c", subcore_axis_name="s")
    run = pl.kernel(
        _mann_body,
        out_type=jax.ShapeDtypeStruct((BATCH, KD), jnp.float32),
        mesh=mesh,
        compiler_params=pltpu.CompilerParams(needs_layout_passes=False),
        scratch_types=[
            pltpu.VMEM((UPW,), jnp.int32),
            pltpu.VMEM((CH, RD), jnp.float32),
            pltpu.VMEM((CH, RD), jnp.float32),
            pltpu.VMEM((UPW, KD), jnp.float32),
            pltpu.VMEM((UPW, KD), jnp.float32),
            pltpu.SemaphoreType.DMA,
            pltpu.SemaphoreType.DMA,
        ],
    )
    return run(user.astype(jnp.int32), query,
               Value.reshape(Value.shape[0], RD))


# speedup vs baseline: 1.4810x; 1.0222x over previous
import jax
import jax.numpy as jnp
from jax import lax
from jax.experimental import pallas as pl
from jax.experimental.pallas import tpu as pltpu
from jax.experimental.pallas import tpu_sc as plsc

BATCH = 4096
SLOTS = 8
KD = 64
RD = SLOTS * KD
LANES = 16
NCHUNK = KD // LANES
NC = 2
NS = 16
NW = NC * NS
UPW = BATCH // NW


def _mann_body(user_hbm, query_hbm, value_hbm, out_hbm,
               idx_v, rows_v, q_v, out_v, sem):
    wid = lax.axis_index("s") * NC + lax.axis_index("c")
    base = wid * UPW

    pltpu.sync_copy(user_hbm.at[pl.ds(base, UPW)], idx_v)
    gather = pltpu.async_copy(value_hbm.at[idx_v], rows_v, sem)
    pltpu.sync_copy(query_hbm.at[pl.ds(base, UPW)], q_v)
    gather.wait()

    def user_body(u, carry):
        q = [q_v[u, pl.ds(c * LANES, LANES)] for c in range(NCHUNK)]
        mk = [[rows_v[u, pl.ds(s * KD + c * LANES, LANES)]
               for c in range(NCHUNK)] for s in range(SLOTS)]

        scores = []
        for s in range(SLOTS):
            acc = mk[s][0] * q[0]
            for c in range(1, NCHUNK):
                acc = acc + mk[s][c] * q[c]
            scores.append(jnp.sum(acc))

        m = scores[0]
        for s in range(1, SLOTS):
            m = jnp.maximum(m, scores[s])
        zeros = jnp.zeros((LANES,), jnp.float32)
        e = [jnp.exp(zeros + (sc - m)) for sc in scores]
        denom = e[0]
        for s in range(1, SLOTS):
            denom = denom + e[s]

        for c in range(NCHUNK):
            acc = e[0] * mk[0][c]
            for s in range(1, SLOTS):
                acc = acc + e[s] * mk[s][c]
            out_v[u, pl.ds(c * LANES, LANES)] = acc / denom
        return carry

    lax.fori_loop(0, UPW, user_body, 0)
    pltpu.sync_copy(out_v, out_hbm.at[pl.ds(base, UPW)])


def kernel(user, query, Value):
    mesh = plsc.VectorSubcoreMesh(core_axis_name="c", subcore_axis_name="s")
    run = pl.kernel(
        _mann_body,
        out_type=jax.ShapeDtypeStruct((BATCH, KD), jnp.float32),
        mesh=mesh,
        compiler_params=pltpu.CompilerParams(needs_layout_passes=False),
        scratch_types=[
            pltpu.VMEM((UPW,), jnp.int32),
            pltpu.VMEM((UPW, RD), jnp.float32),
            pltpu.VMEM((UPW, KD), jnp.float32),
            pltpu.VMEM((UPW, KD), jnp.float32),
            pltpu.SemaphoreType.DMA,
        ],
    )
    return run(user.astype(jnp.int32), query,
               Value.reshape(Value.shape[0], RD))


# speedup vs baseline: 1.4930x; 1.0081x over previous
import jax
import jax.numpy as jnp
from jax import lax
from jax.experimental import pallas as pl
from jax.experimental.pallas import tpu as pltpu
from jax.experimental.pallas import tpu_sc as plsc

BATCH = 4096
SLOTS = 8
KD = 64
RD = SLOTS * KD
LANES = 16
NCHUNK = KD // LANES
NC = 2
NS = 16
NW = NC * NS
UPW = BATCH // NW


def _mann_body(user_hbm, query_hbm, value_hbm, out_hbm,
               idx_v, rows_v, q_v, out_v, sem):
    wid = lax.axis_index("s") * NC + lax.axis_index("c")
    base = wid * UPW

    pltpu.sync_copy(user_hbm.at[pl.ds(base, UPW)], idx_v)
    gather = pltpu.async_copy(value_hbm.at[idx_v], rows_v, sem)
    pltpu.sync_copy(query_hbm.at[pl.ds(base, UPW)], q_v)
    gather.wait()

    def user_body(u):
        q = [q_v[u, pl.ds(c * LANES, LANES)] for c in range(NCHUNK)]
        mk = [[rows_v[u, pl.ds(s * KD + c * LANES, LANES)]
               for c in range(NCHUNK)] for s in range(SLOTS)]

        scores = []
        for s in range(SLOTS):
            acc = mk[s][0] * q[0]
            for c in range(1, NCHUNK):
                acc = acc + mk[s][c] * q[c]
            scores.append(jnp.sum(acc))

        m = scores[0]
        for s in range(1, SLOTS):
            m = jnp.maximum(m, scores[s])
        zeros = jnp.zeros((LANES,), jnp.float32)
        e = [jnp.exp(zeros + (sc - m)) for sc in scores]
        denom = e[0]
        for s in range(1, SLOTS):
            denom = denom + e[s]

        for c in range(NCHUNK):
            acc = e[0] * mk[0][c]
            for s in range(1, SLOTS):
                acc = acc + e[s] * mk[s][c]
            out_v[u, pl.ds(c * LANES, LANES)] = acc / denom

    plsc.parallel_loop(0, UPW, unroll=2)(user_body)
    pltpu.sync_copy(out_v, out_hbm.at[pl.ds(base, UPW)])


def kernel(user, query, Value):
    mesh = plsc.VectorSubcoreMesh(core_axis_name="c", subcore_axis_name="s")
    run = pl.kernel(
        _mann_body,
        out_type=jax.ShapeDtypeStruct((BATCH, KD), jnp.float32),
        mesh=mesh,
        compiler_params=pltpu.CompilerParams(needs_layout_passes=False),
        scratch_types=[
            pltpu.VMEM((UPW,), jnp.int32),
            pltpu.VMEM((UPW, RD), jnp.float32),
            pltpu.VMEM((UPW, KD), jnp.float32),
            pltpu.VMEM((UPW, KD), jnp.float32),
            pltpu.SemaphoreType.DMA,
        ],
    )
    return run(user.astype(jnp.int32), query,
               Value.reshape(Value.shape[0], RD))


# speedup vs baseline: 1.5043x; 1.0076x over previous
import jax
import jax.numpy as jnp
from jax import lax
from jax.experimental import pallas as pl
from jax.experimental.pallas import tpu as pltpu
from jax.experimental.pallas import tpu_sc as plsc

BATCH = 4096
SLOTS = 8
KD = 64
RD = SLOTS * KD
LANES = 16
NCHUNK = KD // LANES
NC = 2
NS = 16
NW = NC * NS
UPW = BATCH // NW


def _mann_body(user_hbm, query_hbm, value_hbm, out_hbm,
               idx_v, rows_v, q_v, out_v, sem):
    wid = lax.axis_index("s") * NC + lax.axis_index("c")
    base = wid * UPW

    pltpu.sync_copy(user_hbm.at[pl.ds(base, UPW)], idx_v)
    gather = pltpu.async_copy(value_hbm.at[idx_v], rows_v, sem)
    pltpu.sync_copy(query_hbm.at[pl.ds(base, UPW)], q_v)
    gather.wait()

    def user_body(u):
        q = [q_v[u, pl.ds(c * LANES, LANES)] for c in range(NCHUNK)]
        mk = [[rows_v[u, pl.ds(s * KD + c * LANES, LANES)]
               for c in range(NCHUNK)] for s in range(SLOTS)]

        scores = []
        for s in range(SLOTS):
            acc = mk[s][0] * q[0]
            for c in range(1, NCHUNK):
                acc = acc + mk[s][c] * q[c]
            scores.append(jnp.sum(acc))

        m = scores[0]
        for s in range(1, SLOTS):
            m = jnp.maximum(m, scores[s])
        zeros = jnp.zeros((LANES,), jnp.float32)
        e = [jnp.exp(zeros + (sc - m)) for sc in scores]
        denom = e[0]
        for s in range(1, SLOTS):
            denom = denom + e[s]

        for c in range(NCHUNK):
            acc = e[0] * mk[0][c]
            for s in range(1, SLOTS):
                acc = acc + e[s] * mk[s][c]
            out_v[u, pl.ds(c * LANES, LANES)] = acc / denom

    plsc.parallel_loop(0, UPW, unroll=4)(user_body)
    pltpu.sync_copy(out_v, out_hbm.at[pl.ds(base, UPW)])


def kernel(user, query, Value):
    mesh = plsc.VectorSubcoreMesh(core_axis_name="c", subcore_axis_name="s")
    run = pl.kernel(
        _mann_body,
        out_type=jax.ShapeDtypeStruct((BATCH, KD), jnp.float32),
        mesh=mesh,
        compiler_params=pltpu.CompilerParams(needs_layout_passes=False),
        scratch_types=[
            pltpu.VMEM((UPW,), jnp.int32),
            pltpu.VMEM((UPW, RD), jnp.float32),
            pltpu.VMEM((UPW, KD), jnp.float32),
            pltpu.VMEM((UPW, KD), jnp.float32),
            pltpu.SemaphoreType.DMA,
        ],
    )
    return run(user.astype(jnp.int32), query,
               Value.reshape(Value.shape[0], RD))
